# baseline (speedup 1.0000x reference)
import functools

import jax
import jax.numpy as jnp
from jax import lax
from jax.experimental import pallas as pl
from jax.experimental.pallas import tpu as pltpu
from jax.experimental.pallas import tpu_sc as plsc

E = 8
TOPK = 2
H = 1024
FF = 4096
T = 256
FFB = 1024
NF = FF // FFB

L = 16
NG = T // L
NC = 2


def _logits_t_body(x_ref, wg_ref, out_ref):
    out_ref[...] = jnp.dot(x_ref[...], wg_ref[...],
                           preferred_element_type=jnp.float32).T


def _sc_router_body(logits_hbm, comb_hbm, lv, cv, sem):
    wid = lax.axis_index("s")

    @pl.when(wid < NG)
    def _():
        pltpu.sync_copy(logits_hbm, lv)
        cols = [lv[pl.ds(e * T + wid * L, L)] for e in range(E)]
        m = cols[0]
        for e in range(1, E):
            m = jnp.maximum(m, cols[e])
        ex = [jnp.exp(c - m) for c in cols]
        s = ex[0]
        for e in range(1, E):
            s = s + ex[e]
        probs = [v / s for v in ex]
        m1 = probs[0]
        for e in range(1, E):
            m1 = jnp.maximum(m1, probs[e])
        i1 = jnp.full((L,), E, jnp.int32)
        for e in reversed(range(E)):
            i1 = jnp.where(probs[e] == m1, e, i1)
        p2 = [jnp.where(i1 == e, -1.0, probs[e]) for e in range(E)]
        m2 = p2[0]
        for e in range(1, E):
            m2 = jnp.maximum(m2, p2[e])
        i2 = jnp.full((L,), E, jnp.int32)
        for e in reversed(range(E)):
            i2 = jnp.where(p2[e] == m2, e, i2)
        inv = 1.0 / (m1 + m2)
        for e in range(E):
            cv[pl.ds(e * L, L)] = (jnp.where(i1 == e, m1, 0.0)
                                   + jnp.where(i2 == e, m2, 0.0)) * inv
        copies = [pltpu.async_copy(cv.at[pl.ds(e * L, L)],
                                   comb_hbm.at[pl.ds(e * T + wid * L, L)],
                                   sem)
                  for e in range(E)]
        for c in copies:
            c.wait()


def _moe_body(xb_ref, comb_t_ref, w1_ref, w3_ref, w2_ref, out_ref, comb_ref):
    e = pl.program_id(0)
    f = pl.program_id(1)

    @pl.when(jnp.logical_and(e == 0, f == 0))
    def _():
        out_ref[...] = jnp.zeros((T, H), jnp.float32)
        comb_ref[...] = comb_t_ref[...].T

    xb = xb_ref[...]
    w1b = w1_ref[0].astype(jnp.bfloat16)
    w3b = w3_ref[0].astype(jnp.bfloat16)
    g = jnp.dot(xb, w1b, preferred_element_type=jnp.float32)
    u = jnp.dot(xb, w3b, preferred_element_type=jnp.float32)
    h = (g * jax.nn.sigmoid(g)) * u
    w2b = w2_ref[0].astype(jnp.bfloat16)
    y = jnp.dot(h.astype(jnp.bfloat16), w2b, preferred_element_type=jnp.float32)
    lane = lax.broadcasted_iota(jnp.int32, (T, E), 1)
    comb_col = jnp.sum(jnp.where(lane == e, comb_ref[...], 0.0), axis=-1,
                       keepdims=True)
    out_ref[...] += comb_col * y


@jax.jit
def kernel(hidden_states, w_gate, w1, w3, w2):
    logits_t = pl.pallas_call(
        _logits_t_body,
        in_specs=[pl.BlockSpec((T, H), lambda: (0, 0)),
                  pl.BlockSpec((H, E), lambda: (0, 0))],
        out_specs=pl.BlockSpec((E, T), lambda: (0, 0)),
        out_shape=jax.ShapeDtypeStruct((E, T), jnp.float32),
    )(hidden_states, w_gate)

    sc_router = pl.kernel(
        _sc_router_body,
        out_type=jax.ShapeDtypeStruct((E * T,), jnp.float32),
        mesh=plsc.VectorSubcoreMesh(core_axis_name="c", subcore_axis_name="s", num_cores=1),
        scratch_types=[pltpu.VMEM((E * T,), jnp.float32),
                       pltpu.VMEM((E * L,), jnp.float32),
                       pltpu.SemaphoreType.DMA],
    )
    combine_t = jnp.reshape(sc_router(jnp.reshape(logits_t, (E * T,))), (E, T))

    xb = hidden_states.astype(jnp.bfloat16)
    return pl.pallas_call(
        _moe_body,
        grid=(E, NF),
        in_specs=[
            pl.BlockSpec((T, H), lambda e, f: (0, 0)),
            pl.BlockSpec((E, T), lambda e, f: (0, 0)),
            pl.BlockSpec((1, H, FFB), lambda e, f: (e, 0, f)),
            pl.BlockSpec((1, H, FFB), lambda e, f: (e, 0, f)),
            pl.BlockSpec((1, FFB, H), lambda e, f: (e, f, 0)),
        ],
        out_specs=pl.BlockSpec((T, H), lambda e, f: (0, 0)),
        out_shape=jax.ShapeDtypeStruct((T, H), jnp.float32),
        scratch_shapes=[pltpu.VMEM((T, E), jnp.float32)],
        compiler_params=pltpu.CompilerParams(
            dimension_semantics=("arbitrary", "arbitrary")),
    )(xb, combine_t, w1, w3, w2)


# speedup vs baseline: 1.1838x; 1.1838x over previous
import jax
import jax.numpy as jnp
from jax import lax
from jax.experimental import pallas as pl
from jax.experimental.pallas import tpu as pltpu

E = 8
TOPK = 2
H = 1024
FF = 4096
T = 256
FFB = 1024
NF = FF // FFB


def _moe_body(x_ref, wg_ref, w1_ref, w3_ref, w2_ref, out_ref, comb_ref,
              xb_ref):
    e = pl.program_id(0)
    f = pl.program_id(1)

    @pl.when(jnp.logical_and(e == 0, f == 0))
    def _():
        x = x_ref[...]
        xb_ref[...] = x.astype(jnp.bfloat16)
        logits = jnp.dot(x, wg_ref[...], preferred_element_type=jnp.float32)
        m = jnp.max(logits, axis=-1, keepdims=True)
        ex = jnp.exp(logits - m)
        probs = ex / jnp.sum(ex, axis=-1, keepdims=True)
        lane = lax.broadcasted_iota(jnp.int32, (T, E), 1)
        m1 = jnp.max(probs, axis=-1, keepdims=True)
        i1 = jnp.min(jnp.where(probs == m1, lane, E), axis=-1, keepdims=True)
        probs2 = jnp.where(lane == i1, -1.0, probs)
        m2 = jnp.max(probs2, axis=-1, keepdims=True)
        i2 = jnp.min(jnp.where(probs2 == m2, lane, E), axis=-1, keepdims=True)
        comb = jnp.where(lane == i1, m1, 0.0) + jnp.where(lane == i2, m2, 0.0)
        comb_ref[...] = comb / (m1 + m2)
        out_ref[...] = jnp.zeros((T, H), jnp.float32)

    xb = xb_ref[...]
    w1b = w1_ref[0].astype(jnp.bfloat16)
    w3b = w3_ref[0].astype(jnp.bfloat16)
    g = jnp.dot(xb, w1b, preferred_element_type=jnp.float32)
    u = jnp.dot(xb, w3b, preferred_element_type=jnp.float32)
    h = (g * jax.nn.sigmoid(g)) * u
    w2b = w2_ref[0].astype(jnp.bfloat16)
    y = jnp.dot(h.astype(jnp.bfloat16), w2b, preferred_element_type=jnp.float32)
    lane = lax.broadcasted_iota(jnp.int32, (T, E), 1)
    comb_col = jnp.sum(jnp.where(lane == e, comb_ref[...], 0.0), axis=-1,
                       keepdims=True)
    out_ref[...] += comb_col * y


@jax.jit
def kernel(hidden_states, w_gate, w1, w3, w2):
    return pl.pallas_call(
        _moe_body,
        grid=(E, NF),
        in_specs=[
            pl.BlockSpec((T, H), lambda e, f: (0, 0)),
            pl.BlockSpec((H, E), lambda e, f: (0, 0)),
            pl.BlockSpec((1, H, FFB), lambda e, f: (e, 0, f)),
            pl.BlockSpec((1, H, FFB), lambda e, f: (e, 0, f)),
            pl.BlockSpec((1, FFB, H), lambda e, f: (e, f, 0)),
        ],
        out_specs=pl.BlockSpec((T, H), lambda e, f: (0, 0)),
        out_shape=jax.ShapeDtypeStruct((T, H), jnp.float32),
        scratch_shapes=[pltpu.VMEM((T, E), jnp.float32),
                        pltpu.VMEM((T, H), jnp.bfloat16)],
        compiler_params=pltpu.CompilerParams(
            dimension_semantics=("arbitrary", "arbitrary")),
    )(hidden_states, w_gate, w1, w3, w2)
